# Initial kernel scaffold; baseline (speedup 1.0000x reference)
import jax
import jax.numpy as jnp
from jax.experimental import pallas as pl


def kernel(queries_0, queries_1, queries_2, keys_0, keys_1, keys_2, values_0, values_1, values_2, weight):
    raise NotImplementedError("write your pallas kernel here")


# speedup vs baseline: 25.0762x; 25.0762x over previous
import functools

import jax
import jax.numpy as jnp
from jax import lax
from jax.experimental import pallas as pl
from jax.experimental.pallas import tpu as pltpu
from jax.experimental.pallas import tpu_sc as plsc

_NH = 8
_D = 16
_BH = 16
_SCALE = 0.25
_NEG = -1e30


def _tokens_bh(x):
    b, c, hh, ww = x.shape
    x = x.reshape(b, _NH, _D, hh, ww)
    return x.transpose(0, 1, 3, 4, 2).reshape(b * _NH, hh * ww, _D)


def _s2d_rows(x):
    b, c, hh, ww = x.shape
    x = x.reshape(b, _NH, _D, hh // 2, 2, ww // 2, 2)
    x = x.transpose(0, 1, 3, 5, 4, 6, 2)
    return x.reshape(b * _NH * (hh // 2) * (ww // 2), 4 * _D)


def _q_groups(x):
    b, c, hh, ww = x.shape
    x = x.reshape(b, _NH, _D, hh // 2, 2, ww // 2, 2)
    x = x.transpose(0, 1, 3, 5, 4, 6, 2)
    return x.reshape(b * _NH, (hh // 2) * (ww // 2), 4, _D)


def _wsm(w_ref):
    w0 = w_ref[0]
    w1 = w_ref[1]
    w2 = w_ref[2]
    m = jnp.maximum(jnp.maximum(w0, w1), w2)
    e0 = jnp.exp(w0 - m)
    e1 = jnp.exp(w1 - m)
    e2 = jnp.exp(w2 - m)
    s = e0 + e1 + e2
    return e0 / s, e1 / s, e2 / s


def _coarse_body(q_ref, k_ref, v_ref, msg_ref, gidx_ref):
    bh = pl.program_id(0)
    q = q_ref[0]
    k = k_ref[0]
    v = v_ref[0]
    s = lax.dot_general(q, k, (((1,), (1,)), ((), ())),
                        preferred_element_type=jnp.float32) * _SCALE
    m = jnp.max(s, axis=-1, keepdims=True)
    e = jnp.exp(s - m)
    a = e / jnp.sum(e, axis=-1, keepdims=True)
    msg_ref[0] = lax.dot_general(a, v, (((1,), (0,)), ((), ())),
                                 preferred_element_type=jnp.float32)
    kio = lax.broadcasted_iota(jnp.int32, (256, 256), 1)
    work = a
    cols = []
    for _ in range(16):
        mx = jnp.max(work, axis=-1, keepdims=True)
        idx = jnp.min(jnp.where(work == mx, kio, 512), axis=-1)
        cols.append(idx + bh * 256)
        work = jnp.where(kio == idx[:, None], _NEG, work)
    gidx_ref[0] = jnp.stack(cols, axis=-1)


def _coarse(q2, k2, v2):
    spec = pl.BlockSpec((1, 256, _D), lambda i: (i, 0, 0))
    return pl.pallas_call(
        _coarse_body,
        grid=(_BH,),
        in_specs=[spec, spec, spec],
        out_specs=[spec, spec],
        out_shape=[jax.ShapeDtypeStruct((_BH, 256, _D), jnp.float32),
                   jax.ShapeDtypeStruct((_BH, 256, _D), jnp.int32)],
    )(q2, k2, v2)


def _gather_rows(kvtab, idx):
    B = idx.shape[0]
    NW = 32
    CH = 128
    bpw = B // NW
    nch = bpw // CH
    mesh = plsc.VectorSubcoreMesh(core_axis_name="c", subcore_axis_name="s")

    @functools.partial(
        pl.kernel,
        out_type=jax.ShapeDtypeStruct((B, 128), jnp.float32),
        mesh=mesh,
        scratch_types=[
            pltpu.VMEM((CH,), jnp.int32),
            pltpu.VMEM((CH, 128), jnp.float32),
            pltpu.SemaphoreType.DMA,
        ],
    )
    def gk(kv_hbm, idx_hbm, out_hbm, idx_v, rows_v, sem):
        wid = lax.axis_index("s") * 2 + lax.axis_index("c")
        base = wid * bpw

        def body(i, carry):
            off = base + i * CH
            pltpu.sync_copy(idx_hbm.at[pl.ds(off, CH)], idx_v)
            pltpu.async_copy(kv_hbm.at[idx_v], rows_v, sem).wait()
            pltpu.sync_copy(rows_v, out_hbm.at[pl.ds(off, CH)])
            return carry

        lax.fori_loop(0, nch, body, 0)

    return gk(kvtab, idx)


def _fine_attn(q, kv):
    L = q.shape[0]
    k3 = kv[:, :, :64].reshape(L, 16, 4, _D).reshape(L, 64, _D)
    v3 = kv[:, :, 64:].reshape(L, 16, 4, _D).reshape(L, 64, _D)
    qb = q.astype(jnp.bfloat16)
    kb = k3.astype(jnp.bfloat16)
    s = lax.dot_general(qb, kb, (((2,), (2,)), ((0,), (0,))),
                        preferred_element_type=jnp.float32) * _SCALE
    m = jnp.max(s, axis=-1, keepdims=True)
    e = jnp.exp(s - m)
    a = e / jnp.sum(e, axis=-1, keepdims=True)
    msg = lax.dot_general(a.astype(jnp.bfloat16), v3.astype(jnp.bfloat16),
                          (((2,), (1,)), ((0,), (0,))),
                          preferred_element_type=jnp.float32)
    return a, msg


def _fine1_body(q_ref, kv_ref, msg0_ref, gidx0_ref, w_ref,
                car_ref, gidx1_ref, *, Lc):
    bh = pl.program_id(0)
    a, msg1 = _fine_attn(q_ref[0], kv_ref[0])
    ws0, ws1, _ = _wsm(w_ref)
    car_ref[0] = msg0_ref[0][:, None, :] * ws0 + msg1 * ws1

    lidx = gidx0_ref[0] - bh * 256
    kio = lax.broadcasted_iota(jnp.int32, (Lc, 4, 64), 2)
    pio = lax.broadcasted_iota(jnp.int32, (Lc, 4, 16), 2)
    work = a
    cols = []
    for _ in range(16):
        mx = jnp.max(work, axis=-1, keepdims=True)
        j = jnp.min(jnp.where(work == mx, kio, 128), axis=-1)
        prank = j // 4
        slot = j % 4
        onehot = prank[:, :, None] == pio
        parent = jnp.sum(jnp.where(onehot, lidx[:, None, :], 0), axis=-1)
        tid = (2 * (parent // 16) + slot // 2) * 32 + 2 * (parent % 16) + (slot % 2)
        cols.append(tid + bh * 1024)
        work = jnp.where(kio == j[:, :, None], _NEG, work)
    gidx1_ref[0] = jnp.stack(cols, axis=-1)


def _fine1(qg, kvsel, msg0, gidx0, weight):
    Lc = 64
    nchunk = 256 // Lc
    grid = (_BH, nchunk)
    qspec = pl.BlockSpec((1, Lc, 4, _D), lambda i, j: (i, j, 0, 0))
    kspec = pl.BlockSpec((1, Lc, 16, 128), lambda i, j: (i, j, 0, 0))
    mspec = pl.BlockSpec((1, Lc, _D), lambda i, j: (i, j, 0))
    gspec = pl.BlockSpec((1, Lc, 16), lambda i, j: (i, j, 0))
    wspec = pl.BlockSpec(memory_space=pltpu.SMEM)
    kvsel = kvsel.reshape(_BH, 256, 16, 128)
    return pl.pallas_call(
        functools.partial(_fine1_body, Lc=Lc),
        grid=grid,
        in_specs=[qspec, kspec, mspec, gspec, wspec],
        out_specs=[qspec, qspec],
        out_shape=[jax.ShapeDtypeStruct((_BH, 256, 4, _D), jnp.float32),
                   jax.ShapeDtypeStruct((_BH, 256, 4, _D), jnp.int32)],
    )(qg.reshape(_BH, 256, 4, _D), kvsel, msg0, gidx0, weight)


def _fine0_body(q_ref, kv_ref, car_ref, w_ref, out_ref):
    _, msg2 = _fine_attn(q_ref[0], kv_ref[0])
    _, _, ws2 = _wsm(w_ref)
    out_ref[0] = car_ref[0][:, None, :] + msg2 * ws2


def _fine0(qg, kvsel, carried_tok, weight):
    Lc = 128
    nchunk = 1024 // Lc
    grid = (_BH, nchunk)
    qspec = pl.BlockSpec((1, Lc, 4, _D), lambda i, j: (i, j, 0, 0))
    kspec = pl.BlockSpec((1, Lc, 16, 128), lambda i, j: (i, j, 0, 0))
    cspec = pl.BlockSpec((1, Lc, _D), lambda i, j: (i, j, 0))
    wspec = pl.BlockSpec(memory_space=pltpu.SMEM)
    kvsel = kvsel.reshape(_BH, 1024, 16, 128)
    return pl.pallas_call(
        _fine0_body,
        grid=grid,
        in_specs=[qspec, kspec, cspec, wspec],
        out_specs=qspec,
        out_shape=jax.ShapeDtypeStruct((_BH, 1024, 4, _D), jnp.float32),
    )(qg.reshape(_BH, 1024, 4, _D), kvsel, carried_tok, weight)


def kernel(queries_0, queries_1, queries_2, keys_0, keys_1, keys_2,
           values_0, values_1, values_2, weight):
    b = queries_0.shape[0]

    msg0, gidx0 = _coarse(_tokens_bh(queries_2), _tokens_bh(keys_2),
                          _tokens_bh(values_2))

    kv1 = jnp.concatenate([_s2d_rows(keys_1), _s2d_rows(values_1)], axis=1)
    kvsel1 = _gather_rows(kv1, gidx0.reshape(-1))
    carried1, gidx1 = _fine1(_q_groups(queries_1), kvsel1, msg0, gidx0,
                             weight)
    carried_gs = carried1.reshape(_BH, 1024, _D)

    kv0 = jnp.concatenate([_s2d_rows(keys_0), _s2d_rows(values_0)], axis=1)
    kvsel0 = _gather_rows(kv0, gidx1.reshape(-1))
    qg0 = _q_groups(queries_0).reshape(_BH, 16, 2, 16, 2, 4, _D)
    qg0 = qg0.transpose(0, 1, 3, 2, 4, 5, 6).reshape(_BH, 1024, 4, _D)
    out = _fine0(qg0, kvsel0, carried_gs, weight)

    o = out.reshape(b, _NH, 16, 16, 2, 2, 2, 2, _D)
    o = o.transpose(0, 2, 4, 6, 3, 5, 7, 1, 8)
    return o.reshape(b, 4096, _NH, _D)
